# initial kernel scaffold (unmeasured)
import jax
import jax.numpy as jnp
from jax import lax
from jax.experimental import pallas as pl
from jax.experimental.pallas import tpu as pltpu


def kernel(
    x,
):
    def body(*refs):
        pass

    out_shape = jax.ShapeDtypeStruct(..., jnp.float32)
    return pl.pallas_call(body, out_shape=out_shape)(...)


# baseline (device time: 37171 ns/iter reference)
import functools

import jax
import jax.numpy as jnp
from jax import lax
from jax.experimental import pallas as pl
from jax.experimental.pallas import tpu as pltpu

N_STAGES = 5


def kernel(x):
    m, n = x.shape[3], x.shape[4]

    def body(x_ref, out_ref, recv_buf, send_sems, recv_sems):
        mx = lax.axis_index("x")
        my = lax.axis_index("y")
        mz = lax.axis_index("z")

        partners = [
            (1 - mx, my, mz),
            (mx, my ^ 1, mz),
            (mx, my ^ 2, mz),
            (mx, my, mz ^ 1),
            (mx, my, mz ^ 2),
        ]

        barrier_sem = pltpu.get_barrier_semaphore()
        for p in partners:
            pl.semaphore_signal(
                barrier_sem, inc=1,
                device_id=p, device_id_type=pl.DeviceIdType.MESH,
            )
        pl.semaphore_wait(barrier_sem, N_STAGES)

        out_ref[:, :] = x_ref[0, 0, 0, :, :]

        for s, p in enumerate(partners):
            rdma = pltpu.make_async_remote_copy(
                src_ref=out_ref,
                dst_ref=recv_buf.at[s],
                send_sem=send_sems.at[s],
                recv_sem=recv_sems.at[s],
                device_id=p,
                device_id_type=pl.DeviceIdType.MESH,
            )
            rdma.start()
            rdma.wait()
            out_ref[:, :] = out_ref[:, :] + recv_buf[s, :, :]

        @functools.partial(pl.run_scoped, sem=pltpu.SemaphoreType.REGULAR)
        def _(sem):
            for p in partners:
                pl.semaphore_signal(
                    sem, inc=1,
                    device_id=p, device_id_type=pl.DeviceIdType.MESH,
                )
            pl.semaphore_wait(sem, N_STAGES)

    return pl.pallas_call(
        body,
        out_shape=jax.ShapeDtypeStruct((m, n), x.dtype),
        in_specs=[pl.BlockSpec(memory_space=pltpu.VMEM)],
        out_specs=pl.BlockSpec(memory_space=pltpu.VMEM),
        scratch_shapes=[
            pltpu.VMEM((N_STAGES, m, n), x.dtype),
            pltpu.SemaphoreType.DMA((N_STAGES,)),
            pltpu.SemaphoreType.DMA((N_STAGES,)),
        ],
        compiler_params=pltpu.CompilerParams(collective_id=0),
    )(x)


# device time: 29062 ns/iter; 1.2790x vs baseline; 1.2790x over previous
import functools

import jax
import jax.numpy as jnp
from jax import lax
from jax.experimental import pallas as pl
from jax.experimental.pallas import tpu as pltpu

N_STAGES = 5
K = 4


def kernel(x):
    m, n = x.shape[3], x.shape[4]
    ch = m // K

    def body(x_ref, out_ref, recv_buf, send_sems, recv_sems):
        mx = lax.axis_index("x")
        my = lax.axis_index("y")
        mz = lax.axis_index("z")

        partners = [
            (1 - mx, my, mz),
            (mx, my ^ 1, mz),
            (mx, my ^ 2, mz),
            (mx, my, mz ^ 1),
            (mx, my, mz ^ 2),
        ]

        barrier_sem = pltpu.get_barrier_semaphore()
        for p in partners:
            pl.semaphore_signal(
                barrier_sem, inc=1,
                device_id=p, device_id_type=pl.DeviceIdType.MESH,
            )
        pl.semaphore_wait(barrier_sem, N_STAGES)

        out_ref[:, :] = x_ref[0, 0, 0, :, :]

        def make_rdma(s, c):
            return pltpu.make_async_remote_copy(
                src_ref=out_ref.at[pl.ds(c * ch, ch), :],
                dst_ref=recv_buf.at[s, c],
                send_sem=send_sems.at[s, c],
                recv_sem=recv_sems.at[s, c],
                device_id=partners[s],
                device_id_type=pl.DeviceIdType.MESH,
            )

        rdmas = {}
        for c in range(K):
            rdmas[(0, c)] = make_rdma(0, c)
            rdmas[(0, c)].start()

        for s in range(N_STAGES):
            for c in range(K):
                rdmas[(s, c)].wait()
                out_ref[pl.ds(c * ch, ch), :] = (
                    out_ref[pl.ds(c * ch, ch), :] + recv_buf[s, c]
                )
                if s + 1 < N_STAGES:
                    rdmas[(s + 1, c)] = make_rdma(s + 1, c)
                    rdmas[(s + 1, c)].start()

        @functools.partial(pl.run_scoped, sem=pltpu.SemaphoreType.REGULAR)
        def _(sem):
            for p in partners:
                pl.semaphore_signal(
                    sem, inc=1,
                    device_id=p, device_id_type=pl.DeviceIdType.MESH,
                )
            pl.semaphore_wait(sem, N_STAGES)

    return pl.pallas_call(
        body,
        out_shape=jax.ShapeDtypeStruct((m, n), x.dtype),
        in_specs=[pl.BlockSpec(memory_space=pltpu.VMEM)],
        out_specs=pl.BlockSpec(memory_space=pltpu.VMEM),
        scratch_shapes=[
            pltpu.VMEM((N_STAGES, K, ch, n), x.dtype),
            pltpu.SemaphoreType.DMA((N_STAGES, K)),
            pltpu.SemaphoreType.DMA((N_STAGES, K)),
        ],
        compiler_params=pltpu.CompilerParams(collective_id=0),
    )(x)


# device time: 28050 ns/iter; 1.3252x vs baseline; 1.0361x over previous
import functools

import jax
import jax.numpy as jnp
from jax import lax
from jax.experimental import pallas as pl
from jax.experimental.pallas import tpu as pltpu

N_STAGES = 5
K = 8


def kernel(x):
    m, n = x.shape[3], x.shape[4]
    ch = m // K

    def body(x_ref, out_ref, recv_buf, send_sems, recv_sems):
        mx = lax.axis_index("x")
        my = lax.axis_index("y")
        mz = lax.axis_index("z")

        partners = [
            (1 - mx, my, mz),
            (mx, my ^ 1, mz),
            (mx, my ^ 2, mz),
            (mx, my, mz ^ 1),
            (mx, my, mz ^ 2),
        ]

        barrier_sem = pltpu.get_barrier_semaphore()
        for p in partners:
            pl.semaphore_signal(
                barrier_sem, inc=1,
                device_id=p, device_id_type=pl.DeviceIdType.MESH,
            )
        pl.semaphore_wait(barrier_sem, N_STAGES)

        out_ref[:, :] = x_ref[0, 0, 0, :, :]

        def make_rdma(s, c):
            return pltpu.make_async_remote_copy(
                src_ref=out_ref.at[pl.ds(c * ch, ch), :],
                dst_ref=recv_buf.at[s, c],
                send_sem=send_sems.at[s, c],
                recv_sem=recv_sems.at[s, c],
                device_id=partners[s],
                device_id_type=pl.DeviceIdType.MESH,
            )

        rdmas = {}
        for c in range(K):
            rdmas[(0, c)] = make_rdma(0, c)
            rdmas[(0, c)].start()

        for s in range(N_STAGES):
            for c in range(K):
                rdmas[(s, c)].wait()
                out_ref[pl.ds(c * ch, ch), :] = (
                    out_ref[pl.ds(c * ch, ch), :] + recv_buf[s, c]
                )
                if s + 1 < N_STAGES:
                    rdmas[(s + 1, c)] = make_rdma(s + 1, c)
                    rdmas[(s + 1, c)].start()

        @functools.partial(pl.run_scoped, sem=pltpu.SemaphoreType.REGULAR)
        def _(sem):
            for p in partners:
                pl.semaphore_signal(
                    sem, inc=1,
                    device_id=p, device_id_type=pl.DeviceIdType.MESH,
                )
            pl.semaphore_wait(sem, N_STAGES)

    return pl.pallas_call(
        body,
        out_shape=jax.ShapeDtypeStruct((m, n), x.dtype),
        in_specs=[pl.BlockSpec(memory_space=pltpu.VMEM)],
        out_specs=pl.BlockSpec(memory_space=pltpu.VMEM),
        scratch_shapes=[
            pltpu.VMEM((N_STAGES, K, ch, n), x.dtype),
            pltpu.SemaphoreType.DMA((N_STAGES, K)),
            pltpu.SemaphoreType.DMA((N_STAGES, K)),
        ],
        compiler_params=pltpu.CompilerParams(collective_id=0),
    )(x)


# device time: 21818 ns/iter; 1.7037x vs baseline; 1.2856x over previous
import jax
from jax import lax
from jax.experimental import pallas as pl
from jax.experimental.pallas import tpu as pltpu

N_MID = 4
P = 8


def kernel(x):
    m, n = x.shape[3], x.shape[4]
    half = m // 2
    rows = half // P

    def body(x_ref, out_ref, xrs_buf, mid_buf,
             xrs_send, xrs_recv, mid_send, mid_recv, ag_send, ag_recv):
        mx = lax.axis_index("x")
        my = lax.axis_index("y")
        mz = lax.axis_index("z")

        x_partner = (1 - mx, my, mz)
        mid_partners = [
            (mx, my ^ 1, mz),
            (mx, my, mz ^ 1),
            (mx, my ^ 2, mz),
            (mx, my, mz ^ 2),
        ]

        hb = mx * half
        ob = (1 - mx) * half

        barrier_sem = pltpu.get_barrier_semaphore()
        for p in [x_partner] + mid_partners:
            pl.semaphore_signal(
                barrier_sem, inc=1,
                device_id=p, device_id_type=pl.DeviceIdType.MESH,
            )
        out_ref[:, :] = x_ref[0, 0, 0, :, :]
        pl.semaphore_wait(barrier_sem, 5)

        xrs = []
        for c in range(P):
            r = pltpu.make_async_remote_copy(
                src_ref=out_ref.at[pl.ds(ob + c * rows, rows), :],
                dst_ref=xrs_buf.at[c],
                send_sem=xrs_send.at[c],
                recv_sem=xrs_recv.at[c],
                device_id=x_partner,
                device_id_type=pl.DeviceIdType.MESH,
            )
            r.start()
            xrs.append(r)

        def make_mid(p, t):
            return pltpu.make_async_remote_copy(
                src_ref=out_ref.at[pl.ds(hb + p * rows, rows), :],
                dst_ref=mid_buf.at[p, t],
                send_sem=mid_send.at[p, t],
                recv_sem=mid_recv.at[p, t],
                device_id=mid_partners[(p + t) % N_MID],
                device_id_type=pl.DeviceIdType.MESH,
            )

        rdmas = {}
        for c in range(P):
            xrs[c].wait()
            out_ref[pl.ds(hb + c * rows, rows), :] = (
                out_ref[pl.ds(hb + c * rows, rows), :] + xrs_buf[c]
            )
            rdmas[(c, 0)] = make_mid(c, 0)
            rdmas[(c, 0)].start()

        ag = []
        for t in range(N_MID):
            for p in range(P):
                rdmas[(p, t)].wait()
                out_ref[pl.ds(hb + p * rows, rows), :] = (
                    out_ref[pl.ds(hb + p * rows, rows), :] + mid_buf[p, t]
                )
                if t + 1 < N_MID:
                    rdmas[(p, t + 1)] = make_mid(p, t + 1)
                    rdmas[(p, t + 1)].start()
                else:
                    r = pltpu.make_async_remote_copy(
                        src_ref=out_ref.at[pl.ds(hb + p * rows, rows), :],
                        dst_ref=out_ref.at[pl.ds(hb + p * rows, rows), :],
                        send_sem=ag_send.at[p],
                        recv_sem=ag_recv.at[p],
                        device_id=x_partner,
                        device_id_type=pl.DeviceIdType.MESH,
                    )
                    r.start()
                    ag.append(r)

        for r in ag:
            r.wait()

    return pl.pallas_call(
        body,
        out_shape=jax.ShapeDtypeStruct((m, n), x.dtype),
        in_specs=[pl.BlockSpec(memory_space=pltpu.VMEM)],
        out_specs=pl.BlockSpec(memory_space=pltpu.VMEM),
        scratch_shapes=[
            pltpu.VMEM((P, rows, n), x.dtype),
            pltpu.VMEM((P, N_MID, rows, n), x.dtype),
            pltpu.SemaphoreType.DMA((P,)),
            pltpu.SemaphoreType.DMA((P,)),
            pltpu.SemaphoreType.DMA((P, N_MID)),
            pltpu.SemaphoreType.DMA((P, N_MID)),
            pltpu.SemaphoreType.DMA((P,)),
            pltpu.SemaphoreType.DMA((P,)),
        ],
        compiler_params=pltpu.CompilerParams(collective_id=0),
    )(x)
